# baseline (speedup 1.0000x reference)
import functools

import jax
import jax.numpy as jnp
from jax import lax
from jax.experimental import pallas as pl
from jax.experimental.pallas import tpu as pltpu
from jax.experimental.pallas import tpu_sc as plsc

D_VOCAB = 30522
D_EMBED = 768
MAX_LEN = 512
N_SEG = 3
BATCH = 64
SEQ = 512

NC = 2
NS = 16
NW = NC * NS
N_ROWS = BATCH * SEQ
ROWS_PER_W = N_ROWS // NW
CHUNK = 32
N_CHUNKS = ROWS_PER_W // CHUNK
LANES = 16
D_BLOCKS = D_EMBED // LANES


def _sc_body(tok_table, tokens_flat, seg_flat, pos_table, seg_table,
             out_hbm, idx_v, sidx_v, tok_rows, seg_rows, pos_rows,
             sem_tok, sem_seg):
  wid = lax.axis_index("s") * NC + lax.axis_index("c")
  row0 = wid * ROWS_PER_W

  def chunk_body(c, _):
    base = row0 + c * CHUNK
    s0 = lax.rem(c * CHUNK, MAX_LEN)

    pltpu.sync_copy(tokens_flat.at[pl.ds(base, CHUNK)], idx_v)
    pltpu.sync_copy(seg_flat.at[pl.ds(base, CHUNK)], sidx_v)
    pltpu.async_copy(tok_table.at[idx_v], tok_rows, sem_tok).wait()
    pltpu.async_copy(seg_table.at[sidx_v], seg_rows, sem_seg).wait()
    pltpu.sync_copy(pos_table.at[pl.ds(s0, CHUNK)], pos_rows)

    def row_body(j, _):
      for d in range(D_BLOCKS):
        sl = pl.ds(d * LANES, LANES)
        tok_rows[j, sl] = tok_rows[j, sl] + pos_rows[j, sl] + seg_rows[j, sl]
      return 0

    lax.fori_loop(0, CHUNK, row_body, 0)
    pltpu.sync_copy(tok_rows, out_hbm.at[pl.ds(base, CHUNK)])
    return 0

  lax.fori_loop(0, N_CHUNKS, chunk_body, 0)


@jax.jit
def _embed(tokens_flat, seg_flat, tok_table, pos_table, seg_table):
  mesh = plsc.VectorSubcoreMesh(core_axis_name="c", subcore_axis_name="s")
  f = functools.partial(
      pl.kernel,
      out_type=jax.ShapeDtypeStruct((N_ROWS, D_EMBED), jnp.float32),
      mesh=mesh,
      scratch_types=[
          pltpu.VMEM((CHUNK,), jnp.int32),
          pltpu.VMEM((CHUNK,), jnp.int32),
          pltpu.VMEM((CHUNK, D_EMBED), jnp.float32),
          pltpu.VMEM((CHUNK, D_EMBED), jnp.float32),
          pltpu.VMEM((CHUNK, D_EMBED), jnp.float32),
          pltpu.SemaphoreType.DMA,
          pltpu.SemaphoreType.DMA,
      ],
  )(_sc_body)
  return f(tok_table, tokens_flat, seg_flat, pos_table, seg_table)


def kernel(tokens, segment_label, tok_table, pos_table, seg_table):
  tokens_flat = tokens.reshape(-1)
  seg_flat = segment_label.reshape(-1)
  out = _embed(tokens_flat, seg_flat, tok_table, pos_table, seg_table)
  return out.reshape(BATCH, SEQ, D_EMBED)


# speedup vs baseline: 2.6608x; 2.6608x over previous
import functools

import jax
import jax.numpy as jnp
from jax import lax
from jax.experimental import pallas as pl
from jax.experimental.pallas import tpu as pltpu
from jax.experimental.pallas import tpu_sc as plsc

D_VOCAB = 30522
D_EMBED = 768
MAX_LEN = 512
N_SEG = 3
BATCH = 64
SEQ = 512

NC = 2
NS = 16
NW = NC * NS
P_PER_W = SEQ // NW
N_ROWS = BATCH * SEQ
LANES = 16
D_BLOCKS = D_EMBED // LANES
NBUF = 4
N_CHUNKS = BATCH


def _sc_body(tok_table, tokens_r, seg_r, pos_table, seg_table,
             out_hbm, tidx, sidx, buf0, buf1, buf2, buf3,
             psum, seg_v, smem_g, sg0, sg1, sg2, sg3, so0, so1, so2, so3):
  bufs = [buf0, buf1, buf2, buf3]
  sem_g = [sg0, sg1, sg2, sg3]
  sem_o = [so0, so1, so2, so3]

  cid = lax.axis_index("c")
  sid = lax.axis_index("s")
  wid = sid * NC + cid
  p0 = wid * P_PER_W
  iota = lax.iota(jnp.int32, LANES)

  pltpu.sync_copy(tokens_r.at[wid], tidx)
  pltpu.sync_copy(seg_r.at[wid], sidx)
  pltpu.sync_copy(seg_table, seg_v)
  pltpu.sync_copy(pos_table.at[pl.ds(p0, P_PER_W)], buf0)

  def prow(jj, _):
    for g in range(N_SEG):
      row = jj * N_SEG + g
      for d in range(D_BLOCKS):
        sl = pl.ds(d * LANES, LANES)
        psum[row, sl] = buf0[jj, sl] + seg_v[g, sl]
    return 0

  lax.fori_loop(0, P_PER_W, prow, 0)

  def gather_desc(b, k):
    return pltpu.make_async_copy(tok_table.at[tidx.at[b]], bufs[k], sem_g[k])

  def scatter_desc(b, k):
    return pltpu.make_async_copy(
        bufs[k], out_hbm.at[pl.ds(b * SEQ + p0, P_PER_W)], sem_o[k])

  gather_desc(0, 0).start()
  gather_desc(1, 1).start()

  def compute(b, k):
    buf = bufs[k]
    gvec = sidx[b, :]
    for jj in range(P_PER_W):
      smem_g[jj] = gvec[jj]

    def row_body(j, _):
      gj = smem_g[j]
      row = N_SEG * j + gj
      for d in range(D_BLOCKS):
        sl = pl.ds(d * LANES, LANES)
        buf[j, sl] = buf[j, sl] + psum[row, sl]
      return 0

    lax.fori_loop(0, P_PER_W, row_body, 0)

  def quad(i, _):
    for k in range(NBUF):
      b = NBUF * i + k
      gather_desc(b, k).wait()
      k2 = (k + 2) % NBUF
      if k < 2:
        @pl.when(i > 0)
        def _():
          scatter_desc(b - 2, k2).wait()
        gather_desc(b + 2, k2).start()
      else:
        scatter_desc(b - 2, k2).wait()

        @pl.when(i < N_CHUNKS // NBUF - 1)
        def _():
          gather_desc(b + 2, k2).start()
      compute(b, k)
      scatter_desc(b, k).start()
    return 0

  lax.fori_loop(0, N_CHUNKS // NBUF, quad, 0)
  scatter_desc(N_CHUNKS - 2, 2).wait()
  scatter_desc(N_CHUNKS - 1, 3).wait()


@jax.jit
def _embed(tokens_r, seg_r, tok_table, pos_table, seg_table):
  mesh = plsc.VectorSubcoreMesh(core_axis_name="c", subcore_axis_name="s")
  f = functools.partial(
      pl.kernel,
      out_type=jax.ShapeDtypeStruct((N_ROWS, D_EMBED), jnp.float32),
      mesh=mesh,
      scratch_types=[
          pltpu.VMEM((BATCH, P_PER_W), jnp.int32),
          pltpu.VMEM((BATCH, P_PER_W), jnp.int32),
          pltpu.VMEM((P_PER_W, D_EMBED), jnp.float32),
          pltpu.VMEM((P_PER_W, D_EMBED), jnp.float32),
          pltpu.VMEM((P_PER_W, D_EMBED), jnp.float32),
          pltpu.VMEM((P_PER_W, D_EMBED), jnp.float32),
          pltpu.VMEM((P_PER_W * N_SEG, D_EMBED), jnp.float32),
          pltpu.VMEM((N_SEG, D_EMBED), jnp.float32),
          pltpu.SMEM((P_PER_W,), jnp.int32),
          pltpu.SemaphoreType.DMA,
          pltpu.SemaphoreType.DMA,
          pltpu.SemaphoreType.DMA,
          pltpu.SemaphoreType.DMA,
          pltpu.SemaphoreType.DMA,
          pltpu.SemaphoreType.DMA,
          pltpu.SemaphoreType.DMA,
          pltpu.SemaphoreType.DMA,
      ],
  )(_sc_body)
  return f(tok_table, tokens_r, seg_r, pos_table, seg_table)


def _rearrange_ids(x):
  return x.reshape(BATCH, NW, P_PER_W).transpose(1, 0, 2)


def kernel(tokens, segment_label, tok_table, pos_table, seg_table):
  out = _embed(_rearrange_ids(tokens), _rearrange_ids(segment_label),
               tok_table, pos_table, seg_table)
  return out.reshape(BATCH, SEQ, D_EMBED)
